# baseline (speedup 1.0000x reference)
import functools

import jax
import jax.numpy as jnp
from jax import lax
from jax.experimental import pallas as pl
from jax.experimental.pallas import tpu as pltpu
from jax.experimental.pallas import tpu_sc as plsc

_N = 10000
_E = 320000
_H = 128
_NG = 64
_OUT = 8

_NC = 2
_NS = 16
_L = 16
_NW = _NC * _NS
_EPW = _E // _NW

_BE = 2560
_NB = _E // _BE


def _sc_edge_batch(batch, src):
    mesh = plsc.VectorSubcoreMesh(core_axis_name="c", subcore_axis_name="s")

    @functools.partial(
        pl.kernel,
        out_type=jax.ShapeDtypeStruct((_E,), jnp.int32),
        mesh=mesh,
        scratch_types=[
            pltpu.VMEM((_N,), jnp.int32),
            pltpu.VMEM((_EPW,), jnp.int32),
            pltpu.VMEM((_EPW,), jnp.int32),
        ],
        compiler_params=pltpu.CompilerParams(needs_layout_passes=False),
    )
    def k(batch_hbm, src_hbm, out_hbm, tbl_v, idx_v, out_v):
        wid = lax.axis_index("s") * _NC + lax.axis_index("c")
        base = wid * _EPW
        pltpu.sync_copy(batch_hbm, tbl_v)
        pltpu.sync_copy(src_hbm.at[pl.ds(base, _EPW)], idx_v)

        def body(i, carry):
            o = i * _L
            idx = idx_v[pl.ds(o, _L)]
            out_v[pl.ds(o, _L)] = plsc.load_gather(tbl_v, [idx])
            return carry

        lax.fori_loop(0, _EPW // _L, body, 0)
        pltpu.sync_copy(out_v, out_hbm.at[pl.ds(base, _EPW)])

    return k(batch, src)


def _tc_body(attr_ref, seg_ref, Ww, bw, Wel, bel, ete, Wc, bc, Wep, bep,
             pW1, pb1, pW2, pb2, rW1, rb1, rW2, rb2, out_ref, acc_ref):
    i = pl.program_id(0)
    attr = attr_ref[...]
    f32 = jnp.float32
    bf16 = jnp.bfloat16

    e0 = jnp.sin(attr[:, 0:1] * Ww[...] + bw[...]).astype(bf16)
    e1 = jnp.sin(attr[:, 1:2] * Wel[...] + bel[...]).astype(bf16)
    Wcv = Wc[...]
    e3 = jnp.sin(attr[:, 3:4] * Wcv[0:1, :] + attr[:, 4:5] * Wcv[1:2, :]
                 + attr[:, 5:6] * Wcv[2:3, :] + bc[...]).astype(bf16)
    t = attr[:, 2:3].astype(jnp.int32)
    oh_t = (t == lax.broadcasted_iota(jnp.int32, (_BE, 4), 1)).astype(bf16)

    Wep_v = Wep[...]
    m2 = jnp.dot(ete[...], Wep_v[2 * _H:3 * _H, :],
                 preferred_element_type=f32).astype(bf16)
    ea = (jnp.dot(e0, Wep_v[0:_H, :], preferred_element_type=f32)
          + jnp.dot(e1, Wep_v[_H:2 * _H, :], preferred_element_type=f32)
          + jnp.dot(e3, Wep_v[3 * _H:4 * _H, :], preferred_element_type=f32)
          + jnp.dot(oh_t, m2, preferred_element_type=f32)
          + bep[...])

    h1 = jnp.maximum(jnp.dot(ea.astype(bf16), pW1[...],
                             preferred_element_type=f32) + pb1[...], 0.0)
    h = (jnp.dot(h1.astype(bf16), pW2[...], preferred_element_type=f32)
         + pb2[...]).astype(bf16)

    seg = seg_ref[0, 0, :]
    oh_seg = (lax.broadcasted_iota(jnp.int32, (_NG, _BE), 0)
              == seg[None, :]).astype(bf16)
    contrib = jnp.dot(oh_seg, h, preferred_element_type=f32)

    @pl.when(i == 0)
    def _():
        acc_ref[...] = jnp.zeros_like(acc_ref)

    acc_ref[...] += contrib

    @pl.when(i == _NB - 1)
    def _():
        pooled = acc_ref[...]
        g1 = jnp.maximum(jnp.dot(pooled, rW1[...], preferred_element_type=f32)
                         + rb1[...], 0.0)
        g = jnp.dot(g1, rW2[...], preferred_element_type=f32) + rb2[...]
        out_ref[...] = 1.0 / (1.0 + jnp.exp(-g))


def _full(shape):
    return pl.BlockSpec(shape, lambda i: (0,) * len(shape))


def _tc_call(edge_attr, seg3, W_weight, b_weight, W_elayer, b_elayer,
             edge_type_emb, W_convpos, b_convpos, W_eproj, b_eproj,
             phi_W1, phi_b1, phi_W2, phi_b2, rho_W1, rho_b1, rho_W2, rho_b2):
    return pl.pallas_call(
        _tc_body,
        grid=(_NB,),
        in_specs=[
            pl.BlockSpec((_BE, 6), lambda i: (i, 0)),
            pl.BlockSpec((1, 1, _BE), lambda i: (i, 0, 0)),
            _full((1, _H)), _full((_H,)),
            _full((1, _H)), _full((_H,)),
            _full((4, _H)),
            _full((3, _H)), _full((_H,)),
            _full((4 * _H, _H)), _full((_H,)),
            _full((_H, _H)), _full((_H,)),
            _full((_H, _H)), _full((_H,)),
            _full((_H, _H)), _full((_H,)),
            _full((_H, _OUT)), _full((_OUT,)),
        ],
        out_specs=pl.BlockSpec((_NG, _OUT), lambda i: (0, 0)),
        out_shape=jax.ShapeDtypeStruct((_NG, _OUT), jnp.float32),
        scratch_shapes=[pltpu.VMEM((_NG, _H), jnp.float32)],
    )(edge_attr, seg3, W_weight, b_weight, W_elayer, b_elayer,
      edge_type_emb, W_convpos, b_convpos, W_eproj, b_eproj,
      phi_W1, phi_b1, phi_W2, phi_b2, rho_W1, rho_b1, rho_W2, rho_b2)


def kernel(x, edge_index, edge_attr, batch, W_node, b_node, W_neuron,
           b_neuron, node_type_emb, W_xproj, b_xproj, W_weight, b_weight,
           W_elayer, b_elayer, edge_type_emb, W_convpos, b_convpos, W_eproj,
           b_eproj, phi_W1, phi_b1, phi_W2, phi_b2, rho_W1, rho_b1, rho_W2,
           rho_b2):
    src = edge_index[0]
    seg = _sc_edge_batch(batch, src)
    seg3 = seg.reshape(_NB, 1, _BE)
    bf16 = jnp.bfloat16
    return _tc_call(edge_attr, seg3, W_weight, b_weight, W_elayer, b_elayer,
                    edge_type_emb.astype(bf16), W_convpos, b_convpos,
                    W_eproj.astype(bf16), b_eproj,
                    phi_W1.astype(bf16), phi_b1, phi_W2.astype(bf16), phi_b2,
                    rho_W1, rho_b1, rho_W2, rho_b2)


# speedup vs baseline: 3.1617x; 3.1617x over previous
import functools

import jax
import jax.numpy as jnp
from jax import lax
from jax.experimental import pallas as pl
from jax.experimental.pallas import tpu as pltpu
from jax.experimental.pallas import tpu_sc as plsc

_N = 10000
_E = 320000
_H = 128
_NG = 64
_OUT = 8

_NC = 2
_NS = 16
_L = 16
_NW = _NC * _NS
_EPW = _E // _NW

_BE = 2560
_NB = _E // _BE


def _sc_edge_batch(batch, src):
    mesh = plsc.VectorSubcoreMesh(core_axis_name="c", subcore_axis_name="s")

    @functools.partial(
        pl.kernel,
        out_type=jax.ShapeDtypeStruct((_E,), jnp.int32),
        mesh=mesh,
        scratch_types=[
            pltpu.VMEM((_N,), jnp.int32),
            pltpu.VMEM((_EPW,), jnp.int32),
            pltpu.VMEM((_EPW,), jnp.int32),
        ],
        compiler_params=pltpu.CompilerParams(needs_layout_passes=False),
    )
    def k(batch_hbm, src_hbm, out_hbm, tbl_v, idx_v, out_v):
        wid = lax.axis_index("s") * _NC + lax.axis_index("c")
        base = wid * _EPW
        pltpu.sync_copy(batch_hbm, tbl_v)
        pltpu.sync_copy(src_hbm.at[pl.ds(base, _EPW)], idx_v)

        def body(i, carry):
            o = i * _L
            idx = idx_v[pl.ds(o, _L)]
            out_v[pl.ds(o, _L)] = plsc.load_gather(tbl_v, [idx])
            return carry

        lax.fori_loop(0, _EPW // _L, body, 0)
        pltpu.sync_copy(out_v, out_hbm.at[pl.ds(base, _EPW)])

    return k(batch, src)


_INV_2PI = 0.15915494309189535
_TWO_PI = 6.283185307179586
_S1 = 9.99984587e-01
_S3 = -1.66632582e-01
_S5 = 8.31238293e-03
_S7 = -1.93161822e-04
_S9 = 2.17321007e-06


def _fast_sin(x):
    q = jnp.round(x * _INV_2PI)
    r = x - q * _TWO_PI
    r2 = r * r
    p = _S9
    p = p * r2 + _S7
    p = p * r2 + _S5
    p = p * r2 + _S3
    p = p * r2 + _S1
    return p * r


def _tc_body(attr_ref, seg_ref, Ww, bw, Wel, bel, ete, Wc, bc, Wep, bep,
             pW1, pb1, pW2, pb2, rW1, rb1, rW2, rb2, out_ref, acc_ref):
    i = pl.program_id(0)
    attr = attr_ref[...]
    f32 = jnp.float32
    bf16 = jnp.bfloat16

    e0 = _fast_sin(attr[:, 0:1] * Ww[...] + bw[...]).astype(bf16)
    e1 = _fast_sin(attr[:, 1:2] * Wel[...] + bel[...]).astype(bf16)
    Wcv = Wc[...]
    e3 = _fast_sin(attr[:, 3:4] * Wcv[0:1, :] + attr[:, 4:5] * Wcv[1:2, :]
                 + attr[:, 5:6] * Wcv[2:3, :] + bc[...]).astype(bf16)
    t = attr[:, 2:3].astype(jnp.int32)
    oh_t = (t == lax.broadcasted_iota(jnp.int32, (_BE, 4), 1)).astype(bf16)

    Wep_v = Wep[...]
    m2 = jnp.dot(ete[...], Wep_v[2 * _H:3 * _H, :],
                 preferred_element_type=f32).astype(bf16)
    ea = (jnp.dot(e0, Wep_v[0:_H, :], preferred_element_type=f32)
          + jnp.dot(e1, Wep_v[_H:2 * _H, :], preferred_element_type=f32)
          + jnp.dot(e3, Wep_v[3 * _H:4 * _H, :], preferred_element_type=f32)
          + jnp.dot(oh_t, m2, preferred_element_type=f32)
          + bep[...])

    h1 = jnp.maximum(jnp.dot(ea.astype(bf16), pW1[...],
                             preferred_element_type=f32) + pb1[...], 0.0)
    h = (jnp.dot(h1.astype(bf16), pW2[...], preferred_element_type=f32)
         + pb2[...]).astype(bf16)

    seg = seg_ref[0, 0, :]
    oh_seg = (lax.broadcasted_iota(jnp.int32, (_NG, _BE), 0)
              == seg[None, :]).astype(bf16)
    contrib = jnp.dot(oh_seg, h, preferred_element_type=f32)

    @pl.when(i == 0)
    def _():
        acc_ref[...] = jnp.zeros_like(acc_ref)

    acc_ref[...] += contrib

    @pl.when(i == _NB - 1)
    def _():
        pooled = acc_ref[...]
        g1 = jnp.maximum(jnp.dot(pooled, rW1[...], preferred_element_type=f32)
                         + rb1[...], 0.0)
        g = jnp.dot(g1, rW2[...], preferred_element_type=f32) + rb2[...]
        out_ref[...] = 1.0 / (1.0 + jnp.exp(-g))


def _full(shape):
    return pl.BlockSpec(shape, lambda i: (0,) * len(shape))


def _tc_call(edge_attr, seg3, W_weight, b_weight, W_elayer, b_elayer,
             edge_type_emb, W_convpos, b_convpos, W_eproj, b_eproj,
             phi_W1, phi_b1, phi_W2, phi_b2, rho_W1, rho_b1, rho_W2, rho_b2):
    return pl.pallas_call(
        _tc_body,
        grid=(_NB,),
        in_specs=[
            pl.BlockSpec((_BE, 6), lambda i: (i, 0)),
            pl.BlockSpec((1, 1, _BE), lambda i: (i, 0, 0)),
            _full((1, _H)), _full((_H,)),
            _full((1, _H)), _full((_H,)),
            _full((4, _H)),
            _full((3, _H)), _full((_H,)),
            _full((4 * _H, _H)), _full((_H,)),
            _full((_H, _H)), _full((_H,)),
            _full((_H, _H)), _full((_H,)),
            _full((_H, _H)), _full((_H,)),
            _full((_H, _OUT)), _full((_OUT,)),
        ],
        out_specs=pl.BlockSpec((_NG, _OUT), lambda i: (0, 0)),
        out_shape=jax.ShapeDtypeStruct((_NG, _OUT), jnp.float32),
        scratch_shapes=[pltpu.VMEM((_NG, _H), jnp.float32)],
    )(edge_attr, seg3, W_weight, b_weight, W_elayer, b_elayer,
      edge_type_emb, W_convpos, b_convpos, W_eproj, b_eproj,
      phi_W1, phi_b1, phi_W2, phi_b2, rho_W1, rho_b1, rho_W2, rho_b2)


def kernel(x, edge_index, edge_attr, batch, W_node, b_node, W_neuron,
           b_neuron, node_type_emb, W_xproj, b_xproj, W_weight, b_weight,
           W_elayer, b_elayer, edge_type_emb, W_convpos, b_convpos, W_eproj,
           b_eproj, phi_W1, phi_b1, phi_W2, phi_b2, rho_W1, rho_b1, rho_W2,
           rho_b2):
    src = edge_index[0]
    seg = _sc_edge_batch(batch, src)
    seg3 = seg.reshape(_NB, 1, _BE)
    bf16 = jnp.bfloat16
    return _tc_call(edge_attr, seg3, W_weight, b_weight, W_elayer, b_elayer,
                    edge_type_emb.astype(bf16), W_convpos, b_convpos,
                    W_eproj.astype(bf16), b_eproj,
                    phi_W1.astype(bf16), phi_b1, phi_W2.astype(bf16), phi_b2,
                    rho_W1, rho_b1, rho_W2, rho_b2)


# speedup vs baseline: 3.2106x; 1.0155x over previous
import functools

import jax
import jax.numpy as jnp
from jax import lax
from jax.experimental import pallas as pl
from jax.experimental.pallas import tpu as pltpu
from jax.experimental.pallas import tpu_sc as plsc

_N = 10000
_E = 320000
_H = 128
_NG = 64
_OUT = 8

_NC = 2
_NS = 16
_L = 16
_NW = _NC * _NS
_EPW = _E // _NW

_BE = 2560
_NB = _E // _BE


def _sc_edge_batch(batch, src):
    mesh = plsc.VectorSubcoreMesh(core_axis_name="c", subcore_axis_name="s")

    @functools.partial(
        pl.kernel,
        out_type=jax.ShapeDtypeStruct((_E,), jnp.int32),
        mesh=mesh,
        scratch_types=[
            pltpu.VMEM((_N,), jnp.int32),
            pltpu.VMEM((_EPW,), jnp.int32),
            pltpu.VMEM((_EPW,), jnp.int32),
        ],
        compiler_params=pltpu.CompilerParams(needs_layout_passes=False),
    )
    def k(batch_hbm, src_hbm, out_hbm, tbl_v, idx_v, out_v):
        wid = lax.axis_index("s") * _NC + lax.axis_index("c")
        base = wid * _EPW
        pltpu.sync_copy(batch_hbm, tbl_v)
        pltpu.sync_copy(src_hbm.at[pl.ds(base, _EPW)], idx_v)

        def body(i, carry):
            o = i * _L
            idx = idx_v[pl.ds(o, _L)]
            out_v[pl.ds(o, _L)] = plsc.load_gather(tbl_v, [idx])
            return carry

        lax.fori_loop(0, _EPW // _L, body, 0)
        pltpu.sync_copy(out_v, out_hbm.at[pl.ds(base, _EPW)])

    return k(batch, src)


_INV_2PI = 0.15915494309189535
_TWO_PI = 6.283185307179586
_S1 = 9.99984587e-01
_S3 = -1.66632582e-01
_S5 = 8.31238293e-03
_S7 = -1.93161822e-04
_S9 = 2.17321007e-06


def _fast_sin(x):
    q = jnp.round(x * _INV_2PI)
    r = x - q * _TWO_PI
    r2 = r * r
    p = _S9
    p = p * r2 + _S7
    p = p * r2 + _S5
    p = p * r2 + _S3
    p = p * r2 + _S1
    return p * r


def _tc_body(attr_ref, seg_ref, Ww, bw, Wel, bel, ete, Wc, bc, Wep, bep,
             pW1, pb1, pW2, pb2, rW1, rb1, rW2, rb2, out_ref, acc_ref):
    i = pl.program_id(0)
    attr = attr_ref[...]
    f32 = jnp.float32

    e0 = _fast_sin(attr[:, 0:1] * Ww[...] + bw[...])
    e1 = _fast_sin(attr[:, 1:2] * Wel[...] + bel[...])
    Wcv = Wc[...]
    e3 = _fast_sin(attr[:, 3:4] * Wcv[0:1, :] + attr[:, 4:5] * Wcv[1:2, :]
                   + attr[:, 5:6] * Wcv[2:3, :] + bc[...])
    t = attr[:, 2:3].astype(jnp.int32)
    oh_t = (t == lax.broadcasted_iota(jnp.int32, (_BE, 4), 1)).astype(f32)

    Wep_v = Wep[...]
    m2 = jnp.dot(ete[...], Wep_v[2 * _H:3 * _H, :], preferred_element_type=f32)
    ea = (jnp.dot(e0, Wep_v[0:_H, :], preferred_element_type=f32)
          + jnp.dot(e1, Wep_v[_H:2 * _H, :], preferred_element_type=f32)
          + jnp.dot(e3, Wep_v[3 * _H:4 * _H, :], preferred_element_type=f32)
          + jnp.dot(oh_t, m2, preferred_element_type=f32)
          + bep[...])

    h1 = jnp.maximum(jnp.dot(ea, pW1[...], preferred_element_type=f32)
                     + pb1[...], 0.0)
    h = jnp.dot(h1, pW2[...], preferred_element_type=f32) + pb2[...]

    seg = seg_ref[0, 0, :]
    oh_seg = (lax.broadcasted_iota(jnp.int32, (_NG, _BE), 0)
              == seg[None, :]).astype(f32)
    contrib = jnp.dot(oh_seg, h, preferred_element_type=f32)

    @pl.when(i == 0)
    def _():
        acc_ref[...] = jnp.zeros_like(acc_ref)

    acc_ref[...] += contrib

    @pl.when(i == _NB - 1)
    def _():
        pooled = acc_ref[...]
        g1 = jnp.maximum(jnp.dot(pooled, rW1[...], preferred_element_type=f32)
                         + rb1[...], 0.0)
        g = jnp.dot(g1, rW2[...], preferred_element_type=f32) + rb2[...]
        out_ref[...] = 1.0 / (1.0 + jnp.exp(-g))


def _full(shape):
    return pl.BlockSpec(shape, lambda i: (0,) * len(shape))


def _tc_call(edge_attr, seg3, W_weight, b_weight, W_elayer, b_elayer,
             edge_type_emb, W_convpos, b_convpos, W_eproj, b_eproj,
             phi_W1, phi_b1, phi_W2, phi_b2, rho_W1, rho_b1, rho_W2, rho_b2):
    return pl.pallas_call(
        _tc_body,
        grid=(_NB,),
        in_specs=[
            pl.BlockSpec((_BE, 6), lambda i: (i, 0)),
            pl.BlockSpec((1, 1, _BE), lambda i: (i, 0, 0)),
            _full((1, _H)), _full((_H,)),
            _full((1, _H)), _full((_H,)),
            _full((4, _H)),
            _full((3, _H)), _full((_H,)),
            _full((4 * _H, _H)), _full((_H,)),
            _full((_H, _H)), _full((_H,)),
            _full((_H, _H)), _full((_H,)),
            _full((_H, _H)), _full((_H,)),
            _full((_H, _OUT)), _full((_OUT,)),
        ],
        out_specs=pl.BlockSpec((_NG, _OUT), lambda i: (0, 0)),
        out_shape=jax.ShapeDtypeStruct((_NG, _OUT), jnp.float32),
        scratch_shapes=[pltpu.VMEM((_NG, _H), jnp.float32)],
    )(edge_attr, seg3, W_weight, b_weight, W_elayer, b_elayer,
      edge_type_emb, W_convpos, b_convpos, W_eproj, b_eproj,
      phi_W1, phi_b1, phi_W2, phi_b2, rho_W1, rho_b1, rho_W2, rho_b2)


def kernel(x, edge_index, edge_attr, batch, W_node, b_node, W_neuron,
           b_neuron, node_type_emb, W_xproj, b_xproj, W_weight, b_weight,
           W_elayer, b_elayer, edge_type_emb, W_convpos, b_convpos, W_eproj,
           b_eproj, phi_W1, phi_b1, phi_W2, phi_b2, rho_W1, rho_b1, rho_W2,
           rho_b2):
    src = edge_index[0]
    seg = _sc_edge_batch(batch, src)
    seg3 = seg.reshape(_NB, 1, _BE)
    return _tc_call(edge_attr, seg3, W_weight, b_weight, W_elayer, b_elayer,
                    edge_type_emb, W_convpos, b_convpos, W_eproj, b_eproj,
                    phi_W1, phi_b1, phi_W2, phi_b2,
                    rho_W1, rho_b1, rho_W2, rho_b2)
